# baseline (speedup 1.0000x reference)
import functools

import jax
import jax.numpy as jnp
from jax import lax
from jax.experimental import pallas as pl
from jax.experimental.pallas import tpu as pltpu
from jax.experimental.pallas import tpu_sc as plsc

SEQ = 200
EMBED = 64
HID = 128
GATES = 4 * HID

_NC = 2
_NS = 16
_NW = _NC * _NS
_B_PAD = 256
_B_PER_W = _B_PAD // _NW


def _sc_gather_body(table_hbm, idx_hbm, out_hbm, idx_v, rows_v, sem):
    wid = lax.axis_index("s") * _NC + lax.axis_index("c")
    base = wid * _B_PER_W
    pltpu.sync_copy(idx_hbm.at[pl.ds(base, _B_PER_W)], idx_v)
    pltpu.async_copy(table_hbm.at[idx_v], rows_v, sem).wait()
    pltpu.sync_copy(rows_v, out_hbm.at[pl.ds(base, _B_PER_W)])


def _sc_gather(table, idx):
    mesh = plsc.VectorSubcoreMesh(core_axis_name="c", subcore_axis_name="s")
    f = functools.partial(
        pl.kernel,
        mesh=mesh,
        out_type=jax.ShapeDtypeStruct((_B_PAD, EMBED), jnp.float32),
        scratch_types=[
            pltpu.VMEM((_B_PER_W,), jnp.int32),
            pltpu.VMEM((_B_PER_W, EMBED), jnp.float32),
            pltpu.SemaphoreType.DMA,
        ],
        compiler_params=pltpu.CompilerParams(use_tc_tiling_on_sc=False),
    )(_sc_gather_body)
    return f(table, idx)


def _lstm_body(e_ref, wih_ref, whh_ref, b_ref, wout_ref, bout_ref, x_ref, gx_ref):
    gx_ref[...] = (
        lax.dot_general(
            e_ref[...], wih_ref[...], (((1,), (1,)), ((), ())),
            preferred_element_type=jnp.float32,
        )
        + b_ref[...]
    )
    whh = whh_ref[...]

    def step(t, carry):
        h, c = carry
        g = gx_ref[pl.ds(t, 1), :] + lax.dot_general(
            h, whh, (((1,), (1,)), ((), ())), preferred_element_type=jnp.float32
        )
        i = jax.nn.sigmoid(g[:, 0:HID])
        f = jax.nn.sigmoid(g[:, HID:2 * HID])
        gg = jnp.tanh(g[:, 2 * HID:3 * HID])
        o = jax.nn.sigmoid(g[:, 3 * HID:4 * HID])
        c = f * c + i * gg
        h = o * jnp.tanh(c)
        return (h, c)

    h0 = jnp.zeros((1, HID), jnp.float32)
    c0 = jnp.zeros((1, HID), jnp.float32)
    h, _ = lax.fori_loop(0, SEQ, step, (h0, c0))
    x_ref[...] = (
        lax.dot_general(
            h, wout_ref[...], (((1,), (1,)), ((), ())),
            preferred_element_type=jnp.float32,
        )
        + bout_ref[...]
    )


def _lstm(rows, W_ih, W_hh, b, W_out, b_out2):
    return pl.pallas_call(
        _lstm_body,
        out_shape=jax.ShapeDtypeStruct((1, 2), jnp.float32),
        scratch_shapes=[pltpu.VMEM((SEQ, GATES), jnp.float32)],
    )(rows, W_ih, W_hh, b, W_out, b_out2)


def kernel(inputs, emb, W_ih, W_hh, b_ih, b_hh, W_out, b_out):
    idx = jnp.zeros((_B_PAD,), jnp.int32).at[:SEQ].set(inputs.astype(jnp.int32))
    rows_pad = _sc_gather(emb, idx)
    rows = rows_pad[:SEQ]
    b = (b_ih + b_hh).reshape(1, GATES)
    x = _lstm(rows, W_ih, W_hh, b, W_out, b_out.reshape(1, 2))
    embeddings = rows.reshape(SEQ, 1, EMBED)
    return (x, embeddings)
